# Initial kernel scaffold; baseline (speedup 1.0000x reference)
import functools

import jax
import jax.numpy as jnp
from jax import lax
from jax.experimental import pallas as pl
from jax.experimental.pallas import tpu as pltpu
from jax.experimental.pallas import tpu_sc as plsc

N = 10000
E = 160000
NP = 10240
ZI = N
OI = N + 1
NSUB = 16
CHUNK = 128
NCHUNK = 80
EPAD = NSUB * NCHUNK * CHUNK
ROWS_PER_SUB = NP // NSUB
BR = 1280


def _make_sc_prop(F):
    mesh = plsc.VectorSubcoreMesh(core_axis_name="c", subcore_axis_name="s")

    @functools.partial(
        pl.kernel,
        mesh=mesh,
        out_type=jax.ShapeDtypeStruct((2, NP, F), jnp.float32),
        scratch_types=[
            pltpu.VMEM((NCHUNK, CHUNK), jnp.int32),
            pltpu.VMEM((NCHUNK, CHUNK), jnp.int32),
            pltpu.VMEM((CHUNK, F), jnp.float32),
            pltpu.VMEM((CHUNK, F), jnp.float32),
            pltpu.VMEM_SHARED((NP, F), jnp.float32),
            pltpu.SemaphoreType.DMA,
            pltpu.SemaphoreType.DMA,
        ],
    )
    def sc_prop(u_hbm, gidx_hbm, sidx_hbm, z_hbm, out_hbm,
                gidx_v, sidx_v, buf0, buf1, s_acc, sem0, sem1):
        c = lax.axis_index("c")
        s = lax.axis_index("s")
        rows = pl.ds(s * ROWS_PER_SUB, ROWS_PER_SUB)
        pltpu.sync_copy(gidx_hbm.at[c, s], gidx_v)
        pltpu.sync_copy(sidx_hbm.at[s], sidx_v)
        pltpu.sync_copy(z_hbm.at[rows], s_acc.at[rows])
        plsc.subcore_barrier()

        pltpu.async_copy(u_hbm.at[gidx_v.at[0]], buf0, sem0)
        pltpu.async_copy(u_hbm.at[gidx_v.at[1]], buf1, sem1)

        def pair(i, _):
            j0 = 2 * i
            pltpu.make_async_copy(u_hbm.at[gidx_v.at[j0]], buf0, sem0).wait()
            pltpu.sync_copy(buf0, s_acc.at[sidx_v.at[j0]], add=True)

            @pl.when(j0 + 2 < NCHUNK)
            def _():
                pltpu.async_copy(u_hbm.at[gidx_v.at[j0 + 2]], buf0, sem0)

            j1 = j0 + 1
            pltpu.make_async_copy(u_hbm.at[gidx_v.at[j1]], buf1, sem1).wait()
            pltpu.sync_copy(buf1, s_acc.at[sidx_v.at[j1]], add=True)

            @pl.when(j1 + 2 < NCHUNK)
            def _():
                pltpu.async_copy(u_hbm.at[gidx_v.at[j1 + 2]], buf1, sem1)

            return 0

        lax.fori_loop(0, NCHUNK // 2, pair, 0)
        plsc.subcore_barrier()
        pltpu.sync_copy(s_acc.at[rows], out_hbm.at[c, rows])

    return sc_prop


_sc_prop64 = _make_sc_prop(64)
_sc_prop32 = _make_sc_prop(32)
_SC_PROP = {64: _sc_prop64, 32: _sc_prop32}


def _make_tc_step(F, DOUT, scale):
    D = 2 * F

    def body(s_ref, tpp_ref, acc_ref, w_ref, dis_ref,
             tnew_ref, u_ref, accn_ref):
        d = dis_ref[...]
        t0 = scale * d * s_ref[0] - tpp_ref[:, 0:F]
        t1 = scale * d * s_ref[1] - tpp_ref[:, F:D]
        tnew_ref[:, 0:F] = t0
        tnew_ref[:, F:D] = t1
        u_ref[0, :, :] = d * t0
        u_ref[1, :, :] = d * t1
        accn_ref[...] = acc_ref[...] + (
            jnp.dot(t0, w_ref[0:F, :], preferred_element_type=jnp.float32)
            + jnp.dot(t1, w_ref[F:D, :], preferred_element_type=jnp.float32))

    grid = NP // BR
    return pl.pallas_call(
        body,
        grid=(grid,),
        in_specs=[
            pl.BlockSpec((2, BR, F), lambda i: (0, i, 0)),
            pl.BlockSpec((BR, D), lambda i: (i, 0)),
            pl.BlockSpec((BR, DOUT), lambda i: (i, 0)),
            pl.BlockSpec((D, DOUT), lambda i: (0, 0)),
            pl.BlockSpec((BR, 1), lambda i: (i, 0)),
        ],
        out_specs=[
            pl.BlockSpec((BR, D), lambda i: (i, 0)),
            pl.BlockSpec((2, BR, F), lambda i: (0, i, 0)),
            pl.BlockSpec((BR, DOUT), lambda i: (i, 0)),
        ],
        out_shape=[
            jax.ShapeDtypeStruct((NP, D), jnp.float32),
            jax.ShapeDtypeStruct((2, NP, F), jnp.float32),
            jax.ShapeDtypeStruct((NP, DOUT), jnp.float32),
        ],
    )


def _make_tc_matmul(DIN, DOUT, pre_silu):
    def body(t_ref, w_ref, b_ref, o_ref):
        t = t_ref[...]
        if pre_silu:
            t = t * jax.nn.sigmoid(t)
        o_ref[...] = jnp.dot(t, w_ref[...],
                             preferred_element_type=jnp.float32) + b_ref[...]

    return pl.pallas_call(
        body,
        grid=(NP // BR,),
        in_specs=[
            pl.BlockSpec((BR, DIN), lambda i: (i, 0)),
            pl.BlockSpec((DIN, DOUT), lambda i: (0, 0)),
            pl.BlockSpec((1, DOUT), lambda i: (0, 0)),
        ],
        out_specs=pl.BlockSpec((BR, DOUT), lambda i: (i, 0)),
        out_shape=jax.ShapeDtypeStruct((NP, DOUT), jnp.float32),
    )


def _make_layer_init(D, do_silu):
    F = D // 2

    def body(a_ref, dis_ref, t_ref, u_ref):
        t = a_ref[...]
        if do_silu:
            t = t * jax.nn.sigmoid(t)
        d = dis_ref[...]
        t_ref[...] = t
        u_ref[0, :, :] = d * t[:, 0:F]
        u_ref[1, :, :] = d * t[:, F:D]

    return pl.pallas_call(
        body,
        grid=(NP // BR,),
        in_specs=[
            pl.BlockSpec((BR, D), lambda i: (i, 0)),
            pl.BlockSpec((BR, 1), lambda i: (i, 0)),
        ],
        out_specs=[
            pl.BlockSpec((BR, D), lambda i: (i, 0)),
            pl.BlockSpec((2, BR, F), lambda i: (0, i, 0)),
        ],
        out_shape=[
            jax.ShapeDtypeStruct((NP, D), jnp.float32),
            jax.ShapeDtypeStruct((2, NP, F), jnp.float32),
        ],
    )


def _dis_body(deg_ref, dis_ref):
    deg = deg_ref[...]
    idx = lax.broadcasted_iota(jnp.int32, (NP, 1), 0)
    ok = (deg > 0.0) & (idx < N)
    dis_ref[...] = jnp.where(ok, lax.rsqrt(jnp.maximum(deg, 1e-12)), 0.0)


_dis_kernel = pl.pallas_call(
    _dis_body,
    out_shape=jax.ShapeDtypeStruct((NP, 1), jnp.float32),
)


def _run_layer(T0, U0, Wp, bp, F, DOUT, gidx, sidx, z, dis):
    acc = _make_tc_matmul(2 * F, DOUT, False)(T0, Wp[0], bp)
    K = Wp.shape[0]
    if K == 1:
        return acc
    sc = _SC_PROP[F]
    step1 = _make_tc_step(F, DOUT, -1.0)
    stepk = _make_tc_step(F, DOUT, -2.0)
    Tzero = jnp.zeros((NP, 2 * F), jnp.float32)

    S = sc(U0.reshape(2 * NP, F), gidx, sidx, z)
    T1, U1, acc = step1(S, Tzero, acc, Wp[1], dis)
    if K == 2:
        return acc

    def step(carry, Wk):
        Tp, Tpp, U, a = carry
        Sk = sc(U.reshape(2 * NP, F), gidx, sidx, z)
        Tn, Un, a = stepk(Sk, Tpp, a, Wk, dis)
        return (Tn, Tp, Un, a), None

    (_, _, _, acc), _ = lax.scan(step, (T1, T0, U1, acc), Wp[2:])
    return acc


def kernel(x, edge_index, batch, edge_attr, W1, b1, W2, b2, W3, b3, W4):
    f32 = jnp.float32
    row = edge_index[0]
    col = edge_index[1]
    pad = EPAD - E
    rowp = jnp.concatenate([row, jnp.zeros((pad,), jnp.int32)])
    colp = jnp.concatenate([col, jnp.zeros((pad,), jnp.int32)])
    realm = jnp.arange(EPAD, dtype=jnp.int32) < E
    valid = (rowp != colp) & realm

    g_main = jnp.where(valid, rowp, ZI)
    gidx = jnp.stack([g_main, g_main + NP]).reshape(2, NSUB, NCHUNK, CHUNK)
    sidx = jnp.where(valid, colp, ZI).reshape(NSUB, NCHUNK, CHUNK)

    g_deg = jnp.where(valid, OI, ZI)
    gidx_deg = jnp.stack([g_deg, g_deg + NP]).reshape(2, NSUB, NCHUNK, CHUNK)
    sidx_deg = jnp.where(valid, rowp, ZI).reshape(NSUB, NCHUNK, CHUNK)

    z64 = jnp.zeros((NP, 64), f32)
    z32 = jnp.zeros((NP, 32), f32)
    u_deg = jnp.zeros((2 * NP, 32), f32).at[OI].set(1.0).at[NP + OI].set(1.0)

    S_deg = _sc_prop32(u_deg, gidx_deg, sidx_deg, z32)
    deg = S_deg[0, :, 0:1]
    dis = _dis_kernel(deg)

    W1p = jnp.pad(W1, ((0, 0), (0, 0), (0, 8)))
    b1p = jnp.pad(b1, (0, 8)).reshape(1, 128)
    W2p = jnp.pad(W2, ((0, 0), (0, 8), (0, 4)))
    b2p = jnp.pad(b2, (0, 4)).reshape(1, 64)
    W3p = jnp.pad(W3, ((0, 0), (0, 4), (0, 2)))
    b3p = jnp.pad(b3, (0, 2)).reshape(1, 32)
    W4p = jnp.pad(W4[0], ((0, 2), (0, 0)))
    zb4 = jnp.zeros((1, 128), f32)

    x_pad = jnp.pad(x, ((0, NP - N), (0, 0)))

    T0, U0 = _make_layer_init(128, False)(x_pad, dis)
    acc = _run_layer(T0, U0, W1p, b1p, 64, 128, gidx, sidx, z64, dis)
    T0, U0 = _make_layer_init(128, True)(acc, dis)
    acc = _run_layer(T0, U0, W2p, b2p, 64, 64, gidx, sidx, z64, dis)
    T0, U0 = _make_layer_init(64, True)(acc, dis)
    acc = _run_layer(T0, U0, W3p, b3p, 32, 32, gidx, sidx, z32, dis)
    out = _make_tc_matmul(32, 128, True)(acc, W4p, zb4)
    return out[:N]


# speedup vs baseline: 2.2817x; 2.2817x over previous
import functools

import jax
import jax.numpy as jnp
from jax import lax
from jax.experimental import pallas as pl
from jax.experimental.pallas import tpu as pltpu
from jax.experimental.pallas import tpu_sc as plsc

N = 10000
E = 160000
NP = 10240
ZI = N
OI = N + 1
NSUB = 16
CHUNK = 128
NCHUNK = 40
EPAD = 2 * NSUB * NCHUNK * CHUNK
ROWS_PER_SUB = NP // NSUB
BR = 1280
D = 128
NBUF = 2


def _make_sc_prop():
    mesh = plsc.VectorSubcoreMesh(core_axis_name="c", subcore_axis_name="s")

    @functools.partial(
        pl.kernel,
        mesh=mesh,
        out_type=jax.ShapeDtypeStruct((2, NP, D), jnp.float32),
        scratch_types=[
            pltpu.VMEM((NCHUNK + 1, CHUNK), jnp.int32),
            pltpu.VMEM((NCHUNK, CHUNK), jnp.int32),
            *[pltpu.VMEM((CHUNK, D), jnp.float32) for _ in range(NBUF)],
            pltpu.VMEM_SHARED((NP, D), jnp.float32),
            *[pltpu.SemaphoreType.DMA for _ in range(2 * NBUF)],
        ],
    )
    def sc_prop(u_hbm, gidx_hbm, sidx_hbm, out_hbm,
                gidx_v, sidx_v, *rest):
        bufs = rest[:NBUF]
        s_acc = rest[NBUF]
        gsems = rest[NBUF + 1:2 * NBUF + 1]
        ssems = rest[2 * NBUF + 1:]
        c = lax.axis_index("c")
        s = lax.axis_index("s")
        pltpu.sync_copy(gidx_hbm.at[c, s], gidx_v)
        pltpu.sync_copy(sidx_hbm.at[c, s], sidx_v)
        pltpu.async_copy(u_hbm.at[gidx_v.at[NCHUNK]], bufs[0], gsems[0]).wait()
        for t in range(ROWS_PER_SUB // CHUNK):
            pltpu.sync_copy(
                bufs[0], s_acc.at[pl.ds(s * ROWS_PER_SUB + t * CHUNK, CHUNK)])
        plsc.subcore_barrier()

        for b in range(NBUF):
            pltpu.async_copy(u_hbm.at[gidx_v.at[b]], bufs[b], gsems[b])

        def group(g, _):
            base = g * NBUF
            for b in range(NBUF):
                j = base + b
                pltpu.make_async_copy(
                    u_hbm.at[gidx_v.at[j]], bufs[b], gsems[b]).wait()
                pltpu.async_copy(
                    bufs[b], s_acc.at[sidx_v.at[j]], ssems[b], add=True)
            for b in range(NBUF):
                j = base + b
                pltpu.make_async_copy(
                    bufs[b], s_acc.at[sidx_v.at[j]], ssems[b]).wait()

                @pl.when(j + NBUF < NCHUNK)
                def _():
                    pltpu.async_copy(
                        u_hbm.at[gidx_v.at[j + NBUF]], bufs[b], gsems[b])

            return 0

        lax.fori_loop(0, NCHUNK // NBUF, group, 0)
        plsc.subcore_barrier()
        rows = pl.ds(s * ROWS_PER_SUB, ROWS_PER_SUB)
        pltpu.sync_copy(s_acc.at[rows], out_hbm.at[c, rows])

    return sc_prop


_SC_PROP_CACHE = {}


def _get_sc_prop():
    if "sc" not in _SC_PROP_CACHE:
        _SC_PROP_CACHE["sc"] = _make_sc_prop()
    return _SC_PROP_CACHE["sc"]


def _make_tc_step(DOUT, scale):
    def body(s_ref, tpp_ref, acc_ref, w_ref, dis_ref,
             tnew_ref, u_ref, accn_ref):
        d = dis_ref[...]
        tn = scale * d * (s_ref[0] + s_ref[1]) - tpp_ref[...]
        tnew_ref[...] = tn
        u_ref[...] = d * tn
        accn_ref[...] = acc_ref[...] + jnp.dot(
            tn, w_ref[...], preferred_element_type=jnp.float32)

    return pl.pallas_call(
        body,
        grid=(NP // BR,),
        in_specs=[
            pl.BlockSpec((2, BR, D), lambda i: (0, i, 0)),
            pl.BlockSpec((BR, D), lambda i: (i, 0)),
            pl.BlockSpec((BR, DOUT), lambda i: (i, 0)),
            pl.BlockSpec((D, DOUT), lambda i: (0, 0)),
            pl.BlockSpec((BR, 1), lambda i: (i, 0)),
        ],
        out_specs=[
            pl.BlockSpec((BR, D), lambda i: (i, 0)),
            pl.BlockSpec((BR, D), lambda i: (i, 0)),
            pl.BlockSpec((BR, DOUT), lambda i: (i, 0)),
        ],
        out_shape=[
            jax.ShapeDtypeStruct((NP, D), jnp.float32),
            jax.ShapeDtypeStruct((NP, D), jnp.float32),
            jax.ShapeDtypeStruct((NP, DOUT), jnp.float32),
        ],
    )


def _make_tc_matmul(DIN, DOUT, pre_silu):
    def body(t_ref, w_ref, b_ref, o_ref):
        t = t_ref[...]
        if pre_silu:
            t = t * jax.nn.sigmoid(t)
        o_ref[...] = jnp.dot(t, w_ref[...],
                             preferred_element_type=jnp.float32) + b_ref[...]

    return pl.pallas_call(
        body,
        grid=(NP // BR,),
        in_specs=[
            pl.BlockSpec((BR, DIN), lambda i: (i, 0)),
            pl.BlockSpec((DIN, DOUT), lambda i: (0, 0)),
            pl.BlockSpec((1, DOUT), lambda i: (0, 0)),
        ],
        out_specs=pl.BlockSpec((BR, DOUT), lambda i: (i, 0)),
        out_shape=jax.ShapeDtypeStruct((NP, DOUT), jnp.float32),
    )


def _make_layer_init(DIN, do_silu):
    def body(a_ref, dis_ref, t_ref, u_ref):
        t = a_ref[...]
        if do_silu:
            t = t * jax.nn.sigmoid(t)
        if DIN < D:
            t = jnp.concatenate(
                [t, jnp.zeros((BR, D - DIN), jnp.float32)], axis=1)
        d = dis_ref[...]
        t_ref[...] = t
        u_ref[...] = d * t

    return pl.pallas_call(
        body,
        grid=(NP // BR,),
        in_specs=[
            pl.BlockSpec((BR, DIN), lambda i: (i, 0)),
            pl.BlockSpec((BR, 1), lambda i: (i, 0)),
        ],
        out_specs=[
            pl.BlockSpec((BR, D), lambda i: (i, 0)),
            pl.BlockSpec((BR, D), lambda i: (i, 0)),
        ],
        out_shape=[
            jax.ShapeDtypeStruct((NP, D), jnp.float32),
            jax.ShapeDtypeStruct((NP, D), jnp.float32),
        ],
    )


def _dis_body(deg_ref, dis_ref):
    deg = deg_ref[...]
    idx = lax.broadcasted_iota(jnp.int32, (NP, 1), 0)
    ok = (deg > 0.0) & (idx < N)
    dis_ref[...] = jnp.where(ok, lax.rsqrt(jnp.maximum(deg, 1e-12)), 0.0)


def _make_dis_kernel():
    return pl.pallas_call(
        _dis_body,
        out_shape=jax.ShapeDtypeStruct((NP, 1), jnp.float32),
    )


def _run_layer(T0, U0, Wp, bp, DOUT, gidx, sidx, dis):
    acc = _make_tc_matmul(D, DOUT, False)(T0, Wp[0], bp)
    K = Wp.shape[0]
    if K == 1:
        return acc
    sc = _get_sc_prop()
    step1 = _make_tc_step(DOUT, -1.0)
    stepk = _make_tc_step(DOUT, -2.0)
    Tzero = jnp.zeros((NP, D), jnp.float32)

    S = sc(U0, gidx, sidx)
    T1, U1, acc = step1(S, Tzero, acc, Wp[1], dis)
    if K == 2:
        return acc

    def step(carry, Wk):
        Tp, Tpp, U, a = carry
        Sk = sc(U, gidx, sidx)
        Tn, Un, a = stepk(Sk, Tpp, a, Wk, dis)
        return (Tn, Tp, Un, a), None

    (_, _, _, acc), _ = lax.scan(step, (T1, T0, U1, acc), Wp[2:])
    return acc


def kernel(x, edge_index, batch, edge_attr, W1, b1, W2, b2, W3, b3, W4):
    f32 = jnp.float32
    row = edge_index[0]
    col = edge_index[1]
    pad = EPAD - E
    rowp = jnp.concatenate([row, jnp.zeros((pad,), jnp.int32)])
    colp = jnp.concatenate([col, jnp.zeros((pad,), jnp.int32)])
    realm = jnp.arange(EPAD, dtype=jnp.int32) < E
    valid = (rowp != colp) & realm

    zchunk = jnp.full((2, NSUB, 1, CHUNK), ZI, jnp.int32)

    def to_chunks(idx):
        return idx.reshape(2, NSUB, NCHUNK, CHUNK)

    def gather_layout(idx):
        return jnp.concatenate([to_chunks(idx), zchunk], axis=2)

    gidx = gather_layout(jnp.where(valid, rowp, ZI))
    sidx = to_chunks(jnp.where(valid, colp, ZI))
    gidx_deg = gather_layout(jnp.where(valid, OI, ZI))
    sidx_deg = to_chunks(jnp.where(valid, rowp, ZI))

    u_deg = jnp.zeros((NP, D), f32).at[OI].set(1.0)

    S_deg = _get_sc_prop()(u_deg, gidx_deg, sidx_deg)
    deg = S_deg[0, :, 0:1] + S_deg[1, :, 0:1]
    dis = _make_dis_kernel()(deg)

    W1p = jnp.pad(W1, ((0, 0), (0, 0), (0, 8)))
    b1p = jnp.pad(b1, (0, 8)).reshape(1, 128)
    W2p = jnp.pad(W2, ((0, 0), (0, 8), (0, 4)))
    b2p = jnp.pad(b2, (0, 4)).reshape(1, 64)
    W3p = jnp.pad(W3, ((0, 0), (0, 68), (0, 2)))
    b3p = jnp.pad(b3, (0, 2)).reshape(1, 32)
    W4p = jnp.pad(W4[0], ((0, 2), (0, 0)))
    zb4 = jnp.zeros((1, 128), f32)

    x_pad = jnp.pad(x, ((0, NP - N), (0, 0)))

    T0, U0 = _make_layer_init(128, False)(x_pad, dis)
    acc = _run_layer(T0, U0, W1p, b1p, 128, gidx, sidx, dis)
    T0, U0 = _make_layer_init(128, True)(acc, dis)
    acc = _run_layer(T0, U0, W2p, b2p, 64, gidx, sidx, dis)
    T0, U0 = _make_layer_init(64, True)(acc, dis)
    acc = _run_layer(T0, U0, W3p, b3p, 32, gidx, sidx, dis)
    out = _make_tc_matmul(32, 128, True)(acc, W4p, zb4)
    return out[:N]
